# baseline (speedup 1.0000x reference)
import functools

import jax
import jax.numpy as jnp
from jax import lax
from jax.experimental import pallas as pl
from jax.experimental.pallas import tpu as pltpu

_EPS = 1e-5


def _fused_body(w_ref, gamma_ref, beta_ref, x_ref, o_ref,
                xbf_ref, g_ref, s_ref, wf_ref, shift_ref,
                *, n_pairs, pairs_per_core, m_true):
    c = pl.program_id(0)
    p = pl.program_id(1)
    j = pl.program_id(2)
    c_in = w_ref.shape[1]
    first_pair = c * pairs_per_core

    @pl.when((p == 0) & (j == 0))
    def _():
        g_ref[...] = jnp.zeros_like(g_ref)
        s_ref[...] = jnp.zeros_like(s_ref)

    @pl.when(p == 0)
    def _():
        xb0 = x_ref[0].astype(jnp.bfloat16)
        xb1 = x_ref[1].astype(jnp.bfloat16)
        g_ref[...] += lax.dot_general(
            xb0, xb0, (((1,), (1,)), ((), ())),
            preferred_element_type=jnp.float32)
        g_ref[...] += lax.dot_general(
            xb1, xb1, (((1,), (1,)), ((), ())),
            preferred_element_type=jnp.float32)
        s_ref[...] += (jnp.sum(x_ref[0], axis=1, keepdims=True) +
                       jnp.sum(x_ref[1], axis=1, keepdims=True))

        @pl.when((j >= first_pair) & (j < first_pair + pairs_per_core))
        def _():
            k = j - first_pair
            xbf_ref[2 * k] = xb0
            xbf_ref[2 * k + 1] = xb1

    @pl.when((p == 0) & (j == n_pairs - 1))
    def _():
        w = w_ref[...].astype(jnp.float32)
        ws = jnp.dot(w, jnp.broadcast_to(s_ref[...], (c_in, c_in)),
                     preferred_element_type=jnp.float32)[:, :1]
        mean = ws / m_true
        wg = jnp.dot(w, g_ref[...], preferred_element_type=jnp.float32)
        e_y2 = jnp.sum(wg * w, axis=1, keepdims=True) / m_true
        var = jnp.maximum(e_y2 - mean * mean, 0.0)
        inv = lax.rsqrt(var + _EPS)
        scale = gamma_ref[...] * inv
        shift_ref[...] = beta_ref[...] - mean * scale
        wf_ref[...] = (scale * w).astype(jnp.bfloat16)

    @pl.when(p == 1)
    def _():
        y = jnp.dot(wf_ref[...], xbf_ref[j],
                    preferred_element_type=jnp.float32)
        o_ref[...] = jnp.maximum(y + shift_ref[...], 0.0)


def kernel(x_nchw, weight, gamma, beta):
    N, C_in, H, W = x_nchw.shape
    C_out = weight.shape[0]
    HW = H * W
    M = float(N * HW)
    x3d = x_nchw.reshape(N, C_in, HW)
    g2 = gamma.reshape(C_out, 1).astype(jnp.float32)
    b2 = beta.reshape(C_out, 1).astype(jnp.float32)

    n_pairs = N // 2
    n_cores = 2
    per_core = N // n_cores
    pairs_per_core = n_pairs // n_cores
    x4d = x3d.reshape(n_pairs, 2, C_in, HW)

    body = functools.partial(_fused_body, n_pairs=n_pairs,
                             pairs_per_core=pairs_per_core, m_true=M)

    out3d = pl.pallas_call(
        body,
        out_shape=jax.ShapeDtypeStruct((N, C_out, HW), jnp.float32),
        grid=(n_cores, 2, n_pairs),
        in_specs=[
            pl.BlockSpec((C_out, C_in), lambda c, p, j: (0, 0)),
            pl.BlockSpec((C_out, 1), lambda c, p, j: (0, 0)),
            pl.BlockSpec((C_out, 1), lambda c, p, j: (0, 0)),
            pl.BlockSpec((None, 2, C_in, HW),
                         lambda c, p, j: (j + p * (n_pairs - 1 - j), 0, 0, 0)),
        ],
        out_specs=pl.BlockSpec((None, C_out, HW),
                               lambda c, p, j: (c * per_core + p * j, 0, 0)),
        scratch_shapes=[
            pltpu.VMEM((per_core, C_in, HW), jnp.bfloat16),
            pltpu.VMEM((C_in, C_in), jnp.float32),
            pltpu.VMEM((C_in, 1), jnp.float32),
            pltpu.VMEM((C_out, C_in), jnp.bfloat16),
            pltpu.VMEM((C_out, 1), jnp.float32),
        ],
        compiler_params=pltpu.CompilerParams(
            dimension_semantics=("parallel", "arbitrary", "arbitrary")),
        cost_estimate=pl.CostEstimate(
            flops=2 * N * HW * C_in * (2 * C_in + C_out),
            transcendentals=2 * C_out,
            bytes_accessed=4 * N * HW * (2 * C_in + C_out)),
    )(weight, g2, b2, x4d)

    return out3d.reshape(N, C_out, H, W)


# speedup vs baseline: 1.1753x; 1.1753x over previous
import functools

import jax
import jax.numpy as jnp
from jax import lax
from jax.experimental import pallas as pl
from jax.experimental.pallas import tpu as pltpu

_EPS = 1e-5


def _fused_body(w_ref, gamma_ref, beta_ref, x_ref, o_ref,
                xbf_ref, g_ref, s_ref, wf_ref, shift_ref, *, n, nb, m_true):
    p = pl.program_id(0)
    b = pl.program_id(1)
    c_in = x_ref.shape[1]

    @pl.when((p == 0) & (b == 0))
    def _():
        g_ref[...] = jnp.zeros_like(g_ref)
        s_ref[...] = jnp.zeros_like(s_ref)

    @pl.when(p == 0)
    def _():
        for i in range(nb):
            x = x_ref[i]
            xb = x.astype(jnp.bfloat16)
            xbf_ref[b * nb + i] = xb
            g_ref[...] += lax.dot_general(
                xb, xb, (((1,), (1,)), ((), ())),
                preferred_element_type=jnp.float32)
            s_ref[...] += jnp.sum(x, axis=1, keepdims=True)

    @pl.when((p == 0) & (b == n // nb - 1))
    def _():
        w = w_ref[...].astype(jnp.float32)
        g = g_ref[...]
        s = s_ref[...]
        ws = jnp.dot(w, jnp.broadcast_to(s, (c_in, c_in)),
                     preferred_element_type=jnp.float32)[:, :1]
        mean = ws / m_true
        wg = jnp.dot(w, g, preferred_element_type=jnp.float32)
        e_y2 = jnp.sum(wg * w, axis=1, keepdims=True) / m_true
        var = jnp.maximum(e_y2 - mean * mean, 0.0)
        inv = lax.rsqrt(var + _EPS)
        scale = gamma_ref[...] * inv
        shift_ref[...] = beta_ref[...] - mean * scale
        wf_ref[...] = (scale * w).astype(jnp.bfloat16)

    @pl.when(p == 1)
    def _():
        for i in range(nb):
            y = jnp.dot(wf_ref[...], xbf_ref[b * nb + i],
                        preferred_element_type=jnp.float32)
            o_ref[i] = jnp.maximum(y + shift_ref[...], 0.0)


def kernel(x_nchw, weight, gamma, beta):
    N, C_in, H, W = x_nchw.shape
    C_out = weight.shape[0]
    HW = H * W
    M = float(N * HW)
    x3d = x_nchw.reshape(N, C_in, HW)
    g2 = gamma.reshape(C_out, 1).astype(jnp.float32)
    b2 = beta.reshape(C_out, 1).astype(jnp.float32)

    NB = 2 if N % 2 == 0 else 1
    NP = N // NB
    body = functools.partial(_fused_body, n=N, nb=NB, m_true=M)
    x4d = x3d.reshape(NP, NB, C_in, HW)

    out4d = pl.pallas_call(
        body,
        out_shape=jax.ShapeDtypeStruct((NP, NB, C_out, HW), jnp.float32),
        grid=(2, NP),
        in_specs=[
            pl.BlockSpec((C_out, C_in), lambda p, b: (0, 0)),
            pl.BlockSpec((C_out, 1), lambda p, b: (0, 0)),
            pl.BlockSpec((C_out, 1), lambda p, b: (0, 0)),
            pl.BlockSpec((None, NB, C_in, HW),
                         lambda p, b: (b + p * (NP - 1 - b), 0, 0, 0)),
        ],
        out_specs=pl.BlockSpec((None, NB, C_out, HW),
                               lambda p, b: (p * b, 0, 0, 0)),
        scratch_shapes=[
            pltpu.VMEM((N, C_in, HW), jnp.bfloat16),
            pltpu.VMEM((C_in, C_in), jnp.float32),
            pltpu.VMEM((C_in, 1), jnp.float32),
            pltpu.VMEM((C_out, C_in), jnp.bfloat16),
            pltpu.VMEM((C_out, 1), jnp.float32),
        ],
        compiler_params=pltpu.CompilerParams(
            dimension_semantics=("arbitrary", "arbitrary")),
        cost_estimate=pl.CostEstimate(
            flops=2 * N * HW * C_in * (C_in + C_out), transcendentals=C_out,
            bytes_accessed=4 * N * HW * (C_in + C_out)),
    )(weight, g2, b2, x4d)

    return out4d.reshape(N, C_out, H, W)


# speedup vs baseline: 4.1361x; 3.5192x over previous
import functools

import jax
import jax.numpy as jnp
from jax import lax
from jax.experimental import pallas as pl
from jax.experimental.pallas import tpu as pltpu

_EPS = 1e-5


def _fused_body(wt_ref, gamma_ref, beta_ref, x_ref, o_ref,
                xbf_ref, g_ref, s_ref, wf_ref, shift_ref, *, n, nb, m_true):
    p = pl.program_id(0)
    b = pl.program_id(1)
    c_in = wt_ref.shape[0]

    @pl.when((p == 0) & (b == 0))
    def _():
        g_ref[...] = jnp.zeros_like(g_ref)
        s_ref[...] = jnp.zeros_like(s_ref)

    @pl.when(p == 0)
    def _():
        for i in range(nb):
            x = x_ref[i]
            xb = x.astype(jnp.bfloat16)
            xbf_ref[b * nb + i] = xb
            g_ref[...] += lax.dot_general(
                xb, xb, (((0,), (0,)), ((), ())),
                preferred_element_type=jnp.float32)
            s_ref[...] += jnp.sum(x, axis=0, keepdims=True)

    @pl.when((p == 0) & (b == n // nb - 1))
    def _():
        wt = wt_ref[...].astype(jnp.float32)
        g = g_ref[...]
        sb = jnp.broadcast_to(s_ref[...], (c_in, c_in))
        mean = jnp.dot(sb, wt,
                       preferred_element_type=jnp.float32)[:1] / m_true
        gwt = jnp.dot(g, wt, preferred_element_type=jnp.float32)
        e_y2 = jnp.sum(wt * gwt, axis=0, keepdims=True) / m_true
        var = jnp.maximum(e_y2 - mean * mean, 0.0)
        inv = lax.rsqrt(var + _EPS)
        scale = gamma_ref[...] * inv
        shift_ref[...] = beta_ref[...] - mean * scale
        wf_ref[...] = (wt * scale).astype(jnp.bfloat16)

    @pl.when(p == 1)
    def _():
        for i in range(nb):
            y = jnp.dot(xbf_ref[b * nb + i], wf_ref[...],
                        preferred_element_type=jnp.float32)
            o_ref[i] = jnp.maximum(y + shift_ref[...], 0.0)


def kernel(x_nchw, weight, gamma, beta):
    N, C_in, H, W = x_nchw.shape
    C_out = weight.shape[0]
    HW = H * W
    M = float(N * HW)
    x_t = jnp.swapaxes(x_nchw.reshape(N, C_in, HW), 1, 2)
    wt = weight.T
    g2 = gamma.reshape(1, C_out).astype(jnp.float32)
    b2 = beta.reshape(1, C_out).astype(jnp.float32)

    NB = 2 if N % 2 == 0 else 1
    NP = N // NB
    body = functools.partial(_fused_body, n=N, nb=NB, m_true=M)
    x4d = x_t.reshape(NP, NB, HW, C_in)

    out4d = pl.pallas_call(
        body,
        out_shape=jax.ShapeDtypeStruct((NP, NB, HW, C_out), jnp.float32),
        grid=(2, NP),
        in_specs=[
            pl.BlockSpec((C_in, C_out), lambda p, b: (0, 0)),
            pl.BlockSpec((1, C_out), lambda p, b: (0, 0)),
            pl.BlockSpec((1, C_out), lambda p, b: (0, 0)),
            pl.BlockSpec((None, NB, HW, C_in),
                         lambda p, b: (b + p * (NP - 1 - b), 0, 0, 0)),
        ],
        out_specs=pl.BlockSpec((None, NB, HW, C_out),
                               lambda p, b: (p * b, 0, 0, 0)),
        scratch_shapes=[
            pltpu.VMEM((N, HW, C_in), jnp.bfloat16),
            pltpu.VMEM((C_in, C_in), jnp.float32),
            pltpu.VMEM((1, C_in), jnp.float32),
            pltpu.VMEM((C_in, C_out), jnp.bfloat16),
            pltpu.VMEM((1, C_out), jnp.float32),
        ],
        compiler_params=pltpu.CompilerParams(
            dimension_semantics=("arbitrary", "arbitrary")),
        cost_estimate=pl.CostEstimate(
            flops=2 * N * HW * C_in * (C_in + C_out), transcendentals=C_out,
            bytes_accessed=4 * N * HW * (C_in + C_out)),
    )(wt, g2, b2, x4d)

    out_t = out4d.reshape(N, HW, C_out)
    return jnp.swapaxes(out_t, 1, 2).reshape(N, C_out, H, W)


# speedup vs baseline: 4.3012x; 1.0399x over previous
import functools

import jax
import jax.numpy as jnp
from jax import lax
from jax.experimental import pallas as pl
from jax.experimental.pallas import tpu as pltpu

_EPS = 1e-5


def _fused_body(w_ref, gamma_ref, beta_ref, x_ref, o_ref,
                xbf_ref, g_ref, s_ref, wf_ref, shift_ref, *, n, nb, m_true):
    p = pl.program_id(0)
    b = pl.program_id(1)
    c_in = w_ref.shape[1]

    @pl.when((p == 0) & (b == 0))
    def _():
        g_ref[...] = jnp.zeros_like(g_ref)
        s_ref[...] = jnp.zeros_like(s_ref)

    @pl.when(p == 0)
    def _():
        for i in range(nb):
            x = x_ref[i]
            xb = x.astype(jnp.bfloat16)
            xbf_ref[b * nb + i] = xb
            g_ref[...] += lax.dot_general(
                xb, xb, (((0,), (0,)), ((), ())),
                preferred_element_type=jnp.float32)
            s_ref[...] += jnp.sum(x, axis=0, keepdims=True)

    @pl.when((p == 0) & (b == n // nb - 1))
    def _():
        wt = jnp.transpose(w_ref[...].astype(jnp.float32))
        g = g_ref[...]
        sb = jnp.broadcast_to(s_ref[...], (c_in, c_in))
        mean = jnp.dot(sb, wt,
                       preferred_element_type=jnp.float32)[:1] / m_true
        gwt = jnp.dot(g, wt, preferred_element_type=jnp.float32)
        e_y2 = jnp.sum(wt * gwt, axis=0, keepdims=True) / m_true
        var = jnp.maximum(e_y2 - mean * mean, 0.0)
        inv = lax.rsqrt(var + _EPS)
        scale = gamma_ref[...] * inv
        shift_ref[...] = beta_ref[...] - mean * scale
        wf_ref[...] = (wt * scale).astype(jnp.bfloat16)

    @pl.when(p == 1)
    def _():
        for i in range(nb):
            y = jnp.dot(xbf_ref[b * nb + i], wf_ref[...],
                        preferred_element_type=jnp.float32)
            o_ref[i] = jnp.maximum(y + shift_ref[...], 0.0)


def kernel(x_nchw, weight, gamma, beta):
    N, C_in, H, W = x_nchw.shape
    C_out = weight.shape[0]
    HW = H * W
    M = float(N * HW)
    x_t = jnp.swapaxes(x_nchw.reshape(N, C_in, HW), 1, 2)
    g2 = gamma.reshape(1, C_out).astype(jnp.float32)
    b2 = beta.reshape(1, C_out).astype(jnp.float32)

    NB = 2 if N % 2 == 0 else 1
    NP = N // NB
    body = functools.partial(_fused_body, n=N, nb=NB, m_true=M)
    x4d = x_t.reshape(NP, NB, HW, C_in)

    out4d = pl.pallas_call(
        body,
        out_shape=jax.ShapeDtypeStruct((NP, NB, HW, C_out), jnp.float32),
        grid=(2, NP),
        in_specs=[
            pl.BlockSpec((C_out, C_in), lambda p, b: (0, 0)),
            pl.BlockSpec((1, C_out), lambda p, b: (0, 0)),
            pl.BlockSpec((1, C_out), lambda p, b: (0, 0)),
            pl.BlockSpec((None, NB, HW, C_in),
                         lambda p, b: (b + p * (NP - 1 - b), 0, 0, 0)),
        ],
        out_specs=pl.BlockSpec((None, NB, HW, C_out),
                               lambda p, b: (p * b, 0, 0, 0)),
        scratch_shapes=[
            pltpu.VMEM((N, HW, C_in), jnp.bfloat16),
            pltpu.VMEM((C_in, C_in), jnp.float32),
            pltpu.VMEM((1, C_in), jnp.float32),
            pltpu.VMEM((C_in, C_out), jnp.bfloat16),
            pltpu.VMEM((1, C_out), jnp.float32),
        ],
        compiler_params=pltpu.CompilerParams(
            dimension_semantics=("arbitrary", "arbitrary")),
        cost_estimate=pl.CostEstimate(
            flops=2 * N * HW * C_in * (C_in + C_out), transcendentals=C_out,
            bytes_accessed=4 * N * HW * (C_in + C_out)),
    )(weight, g2, b2, x4d)

    out_t = out4d.reshape(N, HW, C_out)
    return jnp.swapaxes(out_t, 1, 2).reshape(N, C_out, H, W)


# speedup vs baseline: 4.3604x; 1.0138x over previous
import functools

import jax
import jax.numpy as jnp
from jax import lax
from jax.experimental import pallas as pl
from jax.experimental.pallas import tpu as pltpu

_EPS = 1e-5


def _fused_body(w_ref, gamma_ref, beta_ref, x_ref, o_ref,
                xbf_ref, g_ref, s_ref, wf_ref, shift_ref, *, n, nb, m_true):
    p = pl.program_id(0)
    b = pl.program_id(1)
    c_in = w_ref.shape[1]

    @pl.when((p == 0) & (b == 0))
    def _():
        g_ref[...] = jnp.zeros_like(g_ref)
        s_ref[...] = jnp.zeros_like(s_ref)

    @pl.when(p == 0)
    def _():
        hw = x_ref.shape[1]
        x = x_ref[...].reshape(nb * hw, c_in)
        xb = x.astype(jnp.bfloat16)
        xbf_ref[b] = xb
        g_ref[...] += lax.dot_general(
            xb, xb, (((0,), (0,)), ((), ())),
            preferred_element_type=jnp.float32)
        s_ref[...] += jnp.sum(x, axis=0, keepdims=True)

    @pl.when((p == 0) & (b == n // nb - 1))
    def _():
        wt = jnp.transpose(w_ref[...].astype(jnp.float32))
        g = g_ref[...]
        sb = jnp.broadcast_to(s_ref[...], (c_in, c_in))
        mean = jnp.dot(sb, wt,
                       preferred_element_type=jnp.float32)[:1] / m_true
        gwt = jnp.dot(g, wt, preferred_element_type=jnp.float32)
        e_y2 = jnp.sum(wt * gwt, axis=0, keepdims=True) / m_true
        var = jnp.maximum(e_y2 - mean * mean, 0.0)
        inv = lax.rsqrt(var + _EPS)
        scale = gamma_ref[...] * inv
        shift_ref[...] = beta_ref[...] - mean * scale
        wf_ref[...] = (wt * scale).astype(jnp.bfloat16)

    @pl.when(p == 1)
    def _():
        hw = o_ref.shape[1]
        c_out = o_ref.shape[2]
        y = jnp.dot(xbf_ref[b], wf_ref[...],
                    preferred_element_type=jnp.float32)
        y = jnp.maximum(y + shift_ref[...], 0.0)
        o_ref[...] = y.reshape(nb, hw, c_out)


def kernel(x_nchw, weight, gamma, beta):
    N, C_in, H, W = x_nchw.shape
    C_out = weight.shape[0]
    HW = H * W
    M = float(N * HW)
    x_t = jnp.swapaxes(x_nchw.reshape(N, C_in, HW), 1, 2)
    g2 = gamma.reshape(1, C_out).astype(jnp.float32)
    b2 = beta.reshape(1, C_out).astype(jnp.float32)

    NB = 2 if N % 2 == 0 else 1
    NP = N // NB
    body = functools.partial(_fused_body, n=N, nb=NB, m_true=M)
    x4d = x_t.reshape(NP, NB, HW, C_in)

    out4d = pl.pallas_call(
        body,
        out_shape=jax.ShapeDtypeStruct((NP, NB, HW, C_out), jnp.float32),
        grid=(2, NP),
        in_specs=[
            pl.BlockSpec((C_out, C_in), lambda p, b: (0, 0)),
            pl.BlockSpec((1, C_out), lambda p, b: (0, 0)),
            pl.BlockSpec((1, C_out), lambda p, b: (0, 0)),
            pl.BlockSpec((None, NB, HW, C_in),
                         lambda p, b: (b + p * (NP - 1 - b), 0, 0, 0)),
        ],
        out_specs=pl.BlockSpec((None, NB, HW, C_out),
                               lambda p, b: (p * b, 0, 0, 0)),
        scratch_shapes=[
            pltpu.VMEM((NP, NB * HW, C_in), jnp.bfloat16),
            pltpu.VMEM((C_in, C_in), jnp.float32),
            pltpu.VMEM((1, C_in), jnp.float32),
            pltpu.VMEM((C_in, C_out), jnp.bfloat16),
            pltpu.VMEM((1, C_out), jnp.float32),
        ],
        compiler_params=pltpu.CompilerParams(
            dimension_semantics=("arbitrary", "arbitrary")),
        cost_estimate=pl.CostEstimate(
            flops=2 * N * HW * C_in * (C_in + C_out), transcendentals=C_out,
            bytes_accessed=4 * N * HW * (C_in + C_out)),
    )(weight, g2, b2, x4d)

    out_t = out4d.reshape(N, HW, C_out)
    return jnp.swapaxes(out_t, 1, 2).reshape(N, C_out, H, W)
